# Initial kernel scaffold; baseline (speedup 1.0000x reference)
import jax
import jax.numpy as jnp
from jax.experimental import pallas as pl


def kernel(x, connectivity, Wq, Wk, Wv, gamma, beta):
    raise NotImplementedError("write your pallas kernel here")


# speedup vs baseline: 18.3716x; 18.3716x over previous
import functools

import jax
import jax.numpy as jnp
from jax.experimental import pallas as pl
from jax.experimental.pallas import tpu as pltpu

B, A, IN, OUT, TOPK = 8, 2048, 128, 64, 16
SCALE = 8.0
BM = 256
NEG = -1e30


def _gat_kernel(x_ref, conn_ref, wq_ref, wk_ref, wv_ref, gb_ref, out_ref,
                q_scr, k_scr, v_scr, s_scr):
    i = pl.program_id(1)

    @pl.when(i == 0)
    def _():
        xb = x_ref[0]
        q_scr[...] = jax.lax.dot_general(
            xb, wq_ref[...], (((1,), (1,)), ((), ())),
            preferred_element_type=jnp.float32)
        k_scr[...] = jax.lax.dot_general(
            xb, wk_ref[...], (((1,), (1,)), ((), ())),
            preferred_element_type=jnp.float32)
        v_scr[...] = jax.lax.dot_general(
            xb, wv_ref[...], (((1,), (1,)), ((), ())),
            preferred_element_type=jnp.float32)

    qb = q_scr[pl.ds(i * BM, BM), :]
    s = jax.lax.dot_general(
        qb, k_scr[...], (((1,), (1,)), ((), ())),
        preferred_element_type=jnp.float32)
    s = s * (1.0 / SCALE) + conn_ref[0]

    m1 = jnp.max(s, axis=-1, keepdims=True)
    s_scr[...] = s

    def body(_, carry):
        del carry
        w = s_scr[...]
        m = jnp.max(w, axis=-1, keepdims=True)
        s_scr[...] = jnp.where(w == m, NEG, w)
        return 0

    jax.lax.fori_loop(0, TOPK - 1, body, 0)
    t = jnp.max(s_scr[...], axis=-1, keepdims=True)

    w = jnp.where(s >= t, jnp.exp(s - m1), 0.0)
    z = jnp.sum(w, axis=-1, keepdims=True)
    o = jax.lax.dot_general(
        w, v_scr[...], (((1,), (0,)), ((), ())),
        preferred_element_type=jnp.float32)
    o = o / z

    mu = jnp.mean(o, axis=-1, keepdims=True)
    d = o - mu
    var = jnp.mean(d * d, axis=-1, keepdims=True)
    gamma = gb_ref[0:1, :]
    beta = gb_ref[1:2, :]
    out_ref[0] = d * jax.lax.rsqrt(var + 1e-5) * gamma + beta


@jax.jit
def kernel(x, connectivity, Wq, Wk, Wv, gamma, beta):
    gb = jnp.stack([gamma, beta], axis=0)
    grid = (B, A // BM)
    out = pl.pallas_call(
        _gat_kernel,
        grid=grid,
        in_specs=[
            pl.BlockSpec((1, A, IN), lambda b, i: (b, 0, 0)),
            pl.BlockSpec((1, BM, A), lambda b, i: (b, i, 0)),
            pl.BlockSpec((OUT, IN), lambda b, i: (0, 0)),
            pl.BlockSpec((OUT, IN), lambda b, i: (0, 0)),
            pl.BlockSpec((OUT, IN), lambda b, i: (0, 0)),
            pl.BlockSpec((2, OUT), lambda b, i: (0, 0)),
        ],
        out_specs=pl.BlockSpec((1, BM, OUT), lambda b, i: (b, i, 0)),
        out_shape=jax.ShapeDtypeStruct((B, A, OUT), jnp.float32),
        scratch_shapes=[
            pltpu.VMEM((A, OUT), jnp.float32),
            pltpu.VMEM((A, OUT), jnp.float32),
            pltpu.VMEM((A, OUT), jnp.float32),
            pltpu.VMEM((BM, A), jnp.float32),
        ],
        compiler_params=pltpu.CompilerParams(
            dimension_semantics=("arbitrary", "arbitrary"),
        ),
    )(x, connectivity, Wq, Wk, Wv, gb)
    return out
